# baseline (speedup 1.0000x reference)
import functools

import jax
import jax.numpy as jnp
from jax import lax
from jax.experimental import pallas as pl
from jax.experimental.pallas import tpu as pltpu
from jax.experimental.pallas import tpu_sc as plsc

DIM = 32
L = 16
PACK = 128 // DIM

_info = plsc.get_sparse_core_info()
NC, NS = _info.num_cores, _info.num_subcores
NW = NC * NS

CH = 128
DEPTH = 4
LEAD = 2


def _gather_kernel(n_total, n_chunks):
    mesh = plsc.VectorSubcoreMesh(core_axis_name="c", subcore_axis_name="s")

    @functools.partial(
        pl.kernel,
        mesh=mesh,
        out_type=jax.ShapeDtypeStruct((DIM, n_total), jnp.float32),
        scratch_types=[
            pltpu.VMEM((n_chunks, CH), jnp.int32),
            pltpu.VMEM((n_chunks, CH), jnp.int32),
            pltpu.VMEM((DEPTH, CH, 128), jnp.float32),
            pltpu.VMEM((2, DIM, CH), jnp.float32),
            pltpu.VMEM((L,), jnp.int32),
            pltpu.SemaphoreType.DMA,
            pltpu.SemaphoreType.DMA,
        ],
        compiler_params=pltpu.CompilerParams(
            use_tc_tiling_on_sc=True, needs_layout_passes=False),
    )
    def body(ids_hbm, off_hbm, table_hbm, out_hbm, row_v, lane_v, lines_v,
             stage_v, off_v, g_sem, w_sem):
        wid = lax.axis_index("s") * NC + lax.axis_index("c")
        crow0 = wid * n_chunks
        base = crow0 * CH
        pltpu.sync_copy(off_hbm, off_v)
        pltpu.sync_copy(ids_hbm.at[pl.ds(crow0, n_chunks)], row_v)
        off = off_v[...]

        def shift(j, carry):
            for i in range(CH // L):
                s = pl.ds(i * L, L)
                v = row_v[j, s] + off
                row_v[j, s] = v // PACK
                lane_v[j, s] = (v % PACK) * DIM
            return carry

        lax.fori_loop(0, n_chunks, shift, 0)

        lanes = lax.iota(jnp.int32, L)

        def fire(j):
            pltpu.async_copy(
                table_hbm.at[row_v.at[j]], lines_v.at[j % DEPTH], g_sem)

        for j in range(LEAD):
            fire(j)

        def step(j, carry):
            @pl.when(j + LEAD < n_chunks)
            def _():
                fire(j + LEAD)

            lines = lines_v.at[j % DEPTH]
            pltpu.make_async_copy(table_hbm.at[row_v.at[0]], lines,
                                  g_sem).wait()
            stage = stage_v.at[j % 2]

            @pl.when(j >= 2)
            def _():
                pltpu.make_async_copy(
                    stage, out_hbm.at[:, pl.ds(base, CH)], w_sem).wait()

            def extract(i, c):
                pos = lanes + i * L
                wl = lane_v[j, pl.ds(i * L, L)]
                for d in range(DIM):
                    stage[d, pl.ds(i * L, L)] = plsc.load_gather(
                        lines, [pos, wl + d])
                return c

            lax.fori_loop(0, CH // L, extract, 0)
            pltpu.async_copy(
                stage, out_hbm.at[:, pl.ds(base + j * CH, CH)], w_sem)
            return carry

        lax.fori_loop(0, n_chunks, step, 0)
        for _ in range(2):
            pltpu.make_async_copy(
                stage_v.at[0], out_hbm.at[:, pl.ds(base, CH)], w_sem).wait()

    return body


def kernel(input_ids, offsets, W):
    B, S, H = input_ids.shape
    n_total = B * S * H
    n_chunks = n_total // (NW * CH)
    n_rows = W.shape[0]
    ids_2d = input_ids.reshape(n_total // CH, CH)
    off_tiled = jnp.tile(offsets, L // H)
    table_p = W.reshape(n_rows // PACK, PACK * DIM)
    out_phys = _gather_kernel(n_total, n_chunks)(ids_2d, off_tiled, table_p)
    return out_phys.T.reshape(B, S, H, DIM)


# speedup vs baseline: 1.1029x; 1.1029x over previous
import functools

import jax
import jax.numpy as jnp
from jax import lax
from jax.experimental import pallas as pl
from jax.experimental.pallas import tpu as pltpu
from jax.experimental.pallas import tpu_sc as plsc

DIM = 32
L = 16
PACK = 128 // DIM

_info = plsc.get_sparse_core_info()
NC, NS = _info.num_cores, _info.num_subcores
NW = NC * NS

CH = 128
DEPTH = 4
LEAD = 2


def _gather_kernel(B, H, S, n_chunks):
    mesh = plsc.VectorSubcoreMesh(core_axis_name="c", subcore_axis_name="s")

    @functools.partial(
        pl.kernel,
        mesh=mesh,
        out_type=jax.ShapeDtypeStruct((B, H, DIM, S), jnp.float32),
        scratch_types=[
            pltpu.VMEM((n_chunks, CH), jnp.int32),
            pltpu.VMEM((n_chunks, CH), jnp.int32),
            pltpu.VMEM((DEPTH, CH, 128), jnp.float32),
            pltpu.VMEM((2, DIM, CH), jnp.float32),
            pltpu.VMEM((L,), jnp.int32),
            pltpu.SemaphoreType.DMA,
            pltpu.SemaphoreType.DMA,
        ],
        compiler_params=pltpu.CompilerParams(
            use_tc_tiling_on_sc=True, needs_layout_passes=False),
    )
    def body(ids_hbm, off_hbm, table_hbm, out_hbm, row_v, lane_v, lines_v,
             stage_v, off_v, g_sem, w_sem):
        wid = lax.axis_index("s") * NC + lax.axis_index("c")
        b = wid // H
        h = wid % H
        pltpu.sync_copy(off_hbm.at[h], off_v)
        pltpu.sync_copy(ids_hbm.at[b, h], row_v)
        off = off_v[...]

        def shift(j, carry):
            for i in range(CH // L):
                s = pl.ds(i * L, L)
                v = row_v[j, s] + off
                row_v[j, s] = v // PACK
                lane_v[j, s] = (v % PACK) * DIM
            return carry

        lax.fori_loop(0, n_chunks, shift, 0)

        lanes = lax.iota(jnp.int32, L)

        def fire(j):
            pltpu.async_copy(
                table_hbm.at[row_v.at[j]], lines_v.at[j % DEPTH], g_sem)

        for j in range(LEAD):
            fire(j)

        def step(j, carry):
            @pl.when(j + LEAD < n_chunks)
            def _():
                fire(j + LEAD)

            lines = lines_v.at[j % DEPTH]
            pltpu.make_async_copy(table_hbm.at[row_v.at[0]], lines,
                                  g_sem).wait()
            stage = stage_v.at[j % 2]

            @pl.when(j >= 2)
            def _():
                pltpu.make_async_copy(
                    stage, out_hbm.at[b, h, :, pl.ds(0, CH)], w_sem).wait()

            def extract(i, c):
                pos = lanes + i * L
                wl = lane_v[j, pl.ds(i * L, L)]
                for d in range(DIM):
                    stage[d, pl.ds(i * L, L)] = plsc.load_gather(
                        lines, [pos, wl + d])
                return c

            lax.fori_loop(0, CH // L, extract, 0)
            pltpu.async_copy(
                stage, out_hbm.at[b, h, :, pl.ds(j * CH, CH)], w_sem)
            return carry

        lax.fori_loop(0, n_chunks, step, 0)
        for _ in range(2):
            pltpu.make_async_copy(
                stage_v.at[0], out_hbm.at[b, h, :, pl.ds(0, CH)], w_sem).wait()

    return body


def kernel(input_ids, offsets, W):
    B, S, H = input_ids.shape
    n_chunks = S // CH
    n_rows = W.shape[0]
    ids_t = jnp.transpose(input_ids, (0, 2, 1)).reshape(B, H, n_chunks, CH)
    offs_b = jnp.broadcast_to(offsets[:, None], (H, L))
    table_p = W.reshape(n_rows // PACK, PACK * DIM)
    out_phys = _gather_kernel(B, H, S, n_chunks)(ids_t, offs_b, table_p)
    return jnp.transpose(out_phys, (0, 3, 1, 2))


# speedup vs baseline: 1.1031x; 1.0002x over previous
import functools

import jax
import jax.numpy as jnp
from jax import lax
from jax.experimental import pallas as pl
from jax.experimental.pallas import tpu as pltpu
from jax.experimental.pallas import tpu_sc as plsc

DIM = 32
L = 16
PACK = 128 // DIM

_info = plsc.get_sparse_core_info()
NC, NS = _info.num_cores, _info.num_subcores
NW = NC * NS

CH = 128
DEPTH = 6
LEAD = 4


def _gather_kernel(B, H, S, n_chunks):
    mesh = plsc.VectorSubcoreMesh(core_axis_name="c", subcore_axis_name="s")

    @functools.partial(
        pl.kernel,
        mesh=mesh,
        out_type=jax.ShapeDtypeStruct((B, H, DIM, S), jnp.float32),
        scratch_types=[
            pltpu.VMEM((n_chunks, CH), jnp.int32),
            pltpu.VMEM((n_chunks, CH), jnp.int32),
            pltpu.VMEM((DEPTH, CH, 128), jnp.float32),
            pltpu.VMEM((2, DIM, CH), jnp.float32),
            pltpu.VMEM((L,), jnp.int32),
            pltpu.SemaphoreType.DMA,
            pltpu.SemaphoreType.DMA,
        ],
        compiler_params=pltpu.CompilerParams(
            use_tc_tiling_on_sc=True, needs_layout_passes=False),
    )
    def body(ids_hbm, off_hbm, table_hbm, out_hbm, row_v, lane_v, lines_v,
             stage_v, off_v, g_sem, w_sem):
        wid = lax.axis_index("s") * NC + lax.axis_index("c")
        b = wid // H
        h = wid % H
        pltpu.sync_copy(off_hbm.at[h], off_v)
        pltpu.sync_copy(ids_hbm.at[b, h], row_v)
        off = off_v[...]

        def shift(j, carry):
            for i in range(CH // L):
                s = pl.ds(i * L, L)
                v = row_v[j, s] + off
                row_v[j, s] = v // PACK
                lane_v[j, s] = (v % PACK) * DIM
            return carry

        lax.fori_loop(0, n_chunks, shift, 0)

        lanes = lax.iota(jnp.int32, L)

        def fire(j):
            pltpu.async_copy(
                table_hbm.at[row_v.at[j]], lines_v.at[j % DEPTH], g_sem)

        for j in range(LEAD):
            fire(j)

        def step(j, carry):
            @pl.when(j + LEAD < n_chunks)
            def _():
                fire(j + LEAD)

            lines = lines_v.at[j % DEPTH]
            pltpu.make_async_copy(table_hbm.at[row_v.at[0]], lines,
                                  g_sem).wait()
            stage = stage_v.at[j % 2]

            @pl.when(j >= 2)
            def _():
                pltpu.make_async_copy(
                    stage, out_hbm.at[b, h, :, pl.ds(0, CH)], w_sem).wait()

            def extract(i, c):
                pos = lanes + i * L
                wl = lane_v[j, pl.ds(i * L, L)]
                for d in range(DIM):
                    stage[d, pl.ds(i * L, L)] = plsc.load_gather(
                        lines, [pos, wl + d])
                return c

            lax.fori_loop(0, CH // L, extract, 0)
            pltpu.async_copy(
                stage, out_hbm.at[b, h, :, pl.ds(j * CH, CH)], w_sem)
            return carry

        lax.fori_loop(0, n_chunks, step, 0)
        for _ in range(2):
            pltpu.make_async_copy(
                stage_v.at[0], out_hbm.at[b, h, :, pl.ds(0, CH)], w_sem).wait()

    return body


def kernel(input_ids, offsets, W):
    B, S, H = input_ids.shape
    n_chunks = S // CH
    n_rows = W.shape[0]
    ids_t = jnp.transpose(input_ids, (0, 2, 1)).reshape(B, H, n_chunks, CH)
    offs_b = jnp.broadcast_to(offsets[:, None], (H, L))
    table_p = W.reshape(n_rows // PACK, PACK * DIM)
    out_phys = _gather_kernel(B, H, S, n_chunks)(ids_t, offs_b, table_p)
    return jnp.transpose(out_phys, (0, 3, 1, 2))
